# baseline (speedup 1.0000x reference)
import jax
import jax.numpy as jnp
from jax.experimental import pallas as pl
from jax.experimental.pallas import tpu as pltpu

T = 16384
E = 64
D = 64


def _body(x_hbm, o_hbm, sem):
    def copy(j):
        return pltpu.make_async_copy(
            x_hbm.at[:, pl.ds(E - 1 - j, 1)],
            o_hbm.at[:, pl.ds(j, 1)],
            sem)

    for j in range(E):
        copy(j).start()
    for j in range(E):
        copy(j).wait()


def kernel(x):
    return pl.pallas_call(
        _body,
        in_specs=[pl.BlockSpec(memory_space=pltpu.HBM)],
        out_specs=pl.BlockSpec(memory_space=pltpu.HBM),
        out_shape=jax.ShapeDtypeStruct((T, E, D), jnp.float32),
        scratch_shapes=[pltpu.SemaphoreType.DMA],
    )(x)


# speedup vs baseline: 26.9683x; 26.9683x over previous
import jax
import jax.numpy as jnp
from jax.experimental import pallas as pl
from jax.experimental.pallas import tpu as pltpu

T = 16384
E = 64
D = 64
WR = 32
W = 128
BT = 256
N = T // BT
NBUF = 4


def _flip_block(vbuf, obuf, b):
    ridx = 7 - jax.lax.broadcasted_iota(jnp.int32, (BT, 8, W), 1)
    for k in range(WR // 8):
        seg = vbuf[b, :, 8 * k:8 * (k + 1), :]
        seg = jnp.take_along_axis(seg, ridx, axis=1)
        seg = pltpu.roll(seg, W // 2, 2)
        obuf[b, :, WR - 8 * (k + 1):WR - 8 * k, :] = seg


def _body(x_hbm, o_hbm, vbuf, obuf, gsem, wsem):
    def copy_in(i, b):
        return pltpu.make_async_copy(
            x_hbm.at[pl.ds(i * BT, BT)], vbuf.at[b], gsem.at[b])

    def copy_out(i, b):
        return pltpu.make_async_copy(
            obuf.at[b], o_hbm.at[pl.ds(i * BT, BT)], wsem.at[b])

    def step(i, b, first=False, last=False):
        copy_in(i, b).wait()
        if not first:
            copy_out(i - NBUF, b).wait()
        _flip_block(vbuf, obuf, b)
        copy_out(i, b).start()
        if not last:
            copy_in(i + NBUF, b).start()

    for b in range(NBUF):
        copy_in(b, b).start()
    for b in range(NBUF):
        step(b, b, first=True)

    def group(g, carry):
        i = NBUF * g
        for b in range(NBUF):
            step(i + b, b)
        return carry

    jax.lax.fori_loop(1, N // NBUF - 1, group, 0)

    for b in range(NBUF):
        step(N - NBUF + b, b, last=True)
    for b in range(NBUF):
        copy_out(N - NBUF + b, b).wait()


def kernel(x):
    x4 = x.reshape(T, WR, W)
    y4 = pl.pallas_call(
        _body,
        in_specs=[pl.BlockSpec(memory_space=pltpu.HBM)],
        out_specs=pl.BlockSpec(memory_space=pltpu.HBM),
        out_shape=jax.ShapeDtypeStruct((T, WR, W), jnp.float32),
        scratch_shapes=[
            pltpu.VMEM((NBUF, BT, WR, W), jnp.float32),
            pltpu.VMEM((NBUF, BT, WR, W), jnp.float32),
            pltpu.SemaphoreType.DMA((NBUF,)),
            pltpu.SemaphoreType.DMA((NBUF,)),
        ],
    )(x4)
    return y4.reshape(T, E, D)
